# baseline (speedup 1.0000x reference)
import functools

import jax
import jax.numpy as jnp
import numpy as np
from jax import lax
from jax.experimental import pallas as pl
from jax.experimental.pallas import tpu as pltpu
from jax.experimental.pallas import tpu_sc as plsc

VOCAB_SIZE = 1000000
DIM_MODEL = 64
POSITIONAL_ENCODING_ANGLE_BASE = 10000
POSITIONAL_ENCODING_LENGTH = 2048


def _positional_encoding_np(dim_model, angle_base=POSITIONAL_ENCODING_ANGLE_BASE,
                            length=POSITIONAL_ENCODING_LENGTH):
    depth = dim_model / 2
    positions = np.arange(length)[:, np.newaxis]
    depths = np.arange(depth)[np.newaxis, :]
    angle_rates = 1 / angle_base ** depths
    angle_rads = positions * angle_rates
    return np.concatenate([np.sin(angle_rads), np.cos(angle_rads)],
                          axis=-1).astype(np.float32)


_NW = 32
_LANES = 16
_NBUF = 2


@functools.partial(jax.jit, static_argnames=("batch", "seq_len"))
def _sc_embed(idx, pe, table, *, batch, seq_len):
    dim = table.shape[1]
    seq_per_w = batch // _NW
    n_pairs = seq_per_w // _NBUF
    vregs_per_row = dim // _LANES

    mesh = plsc.VectorSubcoreMesh(core_axis_name="c", subcore_axis_name="s")

    @functools.partial(
        pl.kernel,
        out_type=jax.ShapeDtypeStruct((batch, seq_len, dim), jnp.float32),
        mesh=mesh,
        scratch_types=[
            [pltpu.VMEM((seq_len,), jnp.int32) for _ in range(_NBUF)],
            [pltpu.VMEM((seq_len, dim), jnp.float32) for _ in range(_NBUF)],
            [pltpu.VMEM((seq_len, dim), jnp.float32) for _ in range(_NBUF)],
            pltpu.VMEM((seq_len, dim), jnp.float32),
            [pltpu.SemaphoreType.DMA for _ in range(_NBUF)],
            [pltpu.SemaphoreType.DMA for _ in range(_NBUF)],
        ],
        compiler_params=pltpu.CompilerParams(use_tc_tiling_on_sc=False),
    )
    def body(idx_hbm, pe_hbm, table_hbm, out_hbm,
             ibuf, gbuf, stage, pe_v, gsem, ssem):
        wid = lax.axis_index("s") * 2 + lax.axis_index("c")
        seq0 = wid * seq_per_w

        pltpu.sync_copy(pe_hbm, pe_v)

        for b in range(_NBUF):
            pltpu.sync_copy(idx_hbm.at[pl.ds((seq0 + b) * seq_len, seq_len)],
                            ibuf[b])
            pltpu.async_copy(table_hbm.at[ibuf[b]], gbuf[b], gsem[b])

        def pair(i, _):
            for b in range(_NBUF):
                seq = seq0 + i * _NBUF + b
                pltpu.make_async_copy(table_hbm.at[ibuf[b]], gbuf[b],
                                      gsem[b]).wait()
                @pl.when(i >= 1)
                def _():
                    pltpu.make_async_copy(stage[b], out_hbm.at[seq],
                                          ssem[b]).wait()

                def fma_row(r, _):
                    for c in range(vregs_per_row):
                        sl = pl.ds(c * _LANES, _LANES)
                        stage[b][r, sl] = (gbuf[b][r, sl] * jnp.float32(8.0)
                                           + pe_v[r, sl])
                    return 0

                lax.fori_loop(0, seq_len, fma_row, 0)
                pltpu.async_copy(stage[b], out_hbm.at[seq], ssem[b])

                @pl.when(i < n_pairs - 1)
                def _():
                    nxt = seq + _NBUF
                    pltpu.sync_copy(idx_hbm.at[pl.ds(nxt * seq_len, seq_len)],
                                    ibuf[b])
                    pltpu.async_copy(table_hbm.at[ibuf[b]], gbuf[b], gsem[b])
            return 0

        lax.fori_loop(0, n_pairs, pair, 0)

        for b in range(_NBUF):
            last = seq0 + (n_pairs - 1) * _NBUF + b
            pltpu.make_async_copy(stage[b], out_hbm.at[last], ssem[b]).wait()

    return body(idx, pe, table)


_PE_FULL = _positional_encoding_np(DIM_MODEL)


def kernel(x, table):
    batch, seq_len = x.shape
    idx = x.reshape(-1).astype(jnp.int32)
    pe = jnp.asarray(_PE_FULL[:seq_len])
    return _sc_embed(idx, pe, table, batch=batch, seq_len=seq_len)
